# baseline (speedup 1.0000x reference)
import jax
import jax.numpy as jnp
from jax import lax
from jax.experimental import pallas as pl
from jax.experimental.pallas import tpu as pltpu
from jax.experimental.pallas import tpu_sc as plsc

NC, NS = 2, 16
NW = NC * NS
BATCH, SEQ, HID, VOCAB = 4, 8192, 2048, 256

N_SC_BATCH = 1
ROWS_SC = N_SC_BATCH * SEQ
RPW = ROWS_SC // NW
T = 8
NCHUNK = RPW // T

N_TC_BATCH = BATCH - N_SC_BATCH
ROWS_TC = N_TC_BATCH * SEQ
TB = 512
LBLK = SEQ // TB


def _sc_embed(idx_hbm, emb_hbm, pos_hbm, out_hbm,
              idx_v, buf0, buf1, gbuf0, gbuf1,
              gsem0, gsem1, psem0, psem1, osem0, osem1):
    wid = lax.axis_index("s") * NC + lax.axis_index("c")
    base = pl.multiple_of(wid * RPW, RPW)
    pbase = lax.rem(base, SEQ)
    bufs = (buf0, buf1)
    gbufs = (gbuf0, gbuf1)
    gsems = (gsem0, gsem1)
    psems = (psem0, psem1)
    osems = (osem0, osem1)

    pltpu.sync_copy(idx_hbm.at[pl.ds(base, RPW)], idx_v)

    def launch(c, s):
        p0 = pl.multiple_of(pbase + c * T, T)

        @pl.when(c >= 2)
        def _():
            rp = pl.multiple_of(base + (c - 2) * T, T)
            pltpu.make_async_copy(bufs[s], out_hbm.at[pl.ds(rp, T)],
                                  osems[s]).wait()

        pltpu.async_copy(emb_hbm.at[idx_v.at[pl.ds(c * T, T)]],
                         gbufs[s], gsems[s])
        pltpu.async_copy(pos_hbm.at[pl.ds(p0, T)], bufs[s], psems[s])

    def finish(c, s):
        r0 = pl.multiple_of(base + c * T, T)
        p0 = pl.multiple_of(pbase + c * T, T)
        pltpu.make_async_copy(emb_hbm.at[idx_v.at[pl.ds(c * T, T)]],
                              gbufs[s], gsems[s]).wait()
        pltpu.make_async_copy(pos_hbm.at[pl.ds(p0, T)], bufs[s],
                              psems[s]).wait()

        @plsc.parallel_loop(0, HID, step=16, unroll=4)
        def add_body(j):
            for t in range(T):
                plsc.addupdate(bufs[s].at[t, pl.ds(j, 16)],
                               gbufs[s][t, pl.ds(j, 16)])

        pltpu.async_copy(bufs[s], out_hbm.at[pl.ds(r0, T)], osems[s])

    launch(0, 0)

    def body(i, carry):
        c = 2 * i
        launch(c + 1, 1)
        finish(c, 0)
        launch(c + 2, 0)
        finish(c + 1, 1)
        return carry

    lax.fori_loop(0, NCHUNK // 2 - 1, body, 0)

    c = NCHUNK - 2
    launch(c + 1, 1)
    finish(c, 0)
    finish(c + 1, 1)
    pltpu.make_async_copy(bufs[0], out_hbm.at[pl.ds(base + c * T, T)],
                          osems[0]).wait()
    pltpu.make_async_copy(bufs[1], out_hbm.at[pl.ds(base + (c + 1) * T, T)],
                          osems[1]).wait()


def _sc_call(idx_sc, emb_table, pos_table):
    mesh = plsc.VectorSubcoreMesh(
        core_axis_name="c", subcore_axis_name="s",
        num_cores=NC, num_subcores=NS)
    return pl.kernel(
        _sc_embed,
        out_type=jax.ShapeDtypeStruct((ROWS_SC, HID), jnp.float32),
        mesh=mesh,
        scratch_types=[
            pltpu.VMEM((RPW,), jnp.int32),
            pltpu.VMEM((T, HID), jnp.float32),
            pltpu.VMEM((T, HID), jnp.float32),
            pltpu.VMEM((T, HID), jnp.float32),
            pltpu.VMEM((T, HID), jnp.float32),
            pltpu.SemaphoreType.DMA,
            pltpu.SemaphoreType.DMA,
            pltpu.SemaphoreType.DMA,
            pltpu.SemaphoreType.DMA,
            pltpu.SemaphoreType.DMA,
            pltpu.SemaphoreType.DMA,
        ],
    )(idx_sc, emb_table, pos_table)


def _tc_embed(idx_ref, emb_ref, pos_ref, out_ref):
    ids = idx_ref[0, 0, :]
    onehot = (ids[:, None] ==
              lax.broadcasted_iota(jnp.int32, (1, VOCAB), 1)
              ).astype(jnp.bfloat16)
    tok = lax.dot_general(onehot, emb_ref[...],
                          (((1,), (0,)), ((), ())),
                          preferred_element_type=jnp.float32)
    out_ref[...] = tok + pos_ref[...]


def _tc_call(idx_tc, emb_bf16, pos_table):
    return pl.pallas_call(
        _tc_embed,
        grid=(LBLK, N_TC_BATCH),
        in_specs=[
            pl.BlockSpec((1, 1, TB), lambda lb, bb: (bb * LBLK + lb, 0, 0)),
            pl.BlockSpec((VOCAB, HID), lambda lb, bb: (0, 0)),
            pl.BlockSpec((TB, HID), lambda lb, bb: (lb, 0)),
        ],
        out_specs=pl.BlockSpec((TB, HID), lambda lb, bb: (bb * LBLK + lb, 0)),
        out_shape=jax.ShapeDtypeStruct((ROWS_TC, HID), jnp.float32),
    )(idx_tc, emb_bf16, pos_table)


def kernel(input_bytes, emb_table, pos_table):
    idx = input_bytes.reshape(BATCH * SEQ).astype(jnp.int32)
    idx_sc = idx[:ROWS_SC]
    idx_tc = idx[ROWS_SC:].reshape(ROWS_TC // TB, 1, TB)
    emb_bf16 = emb_table.astype(jnp.bfloat16)
    sc_out = _sc_call(idx_sc, emb_table, pos_table)
    tc_out = _tc_call(idx_tc, emb_bf16, pos_table)
    out = jnp.concatenate([sc_out, tc_out], axis=0)
    return out.reshape(BATCH, SEQ, HID)


# speedup vs baseline: 1.6189x; 1.6189x over previous
import jax
import jax.numpy as jnp
from jax import lax
from jax.experimental import pallas as pl
from jax.experimental.pallas import tpu as pltpu
from jax.experimental.pallas import tpu_sc as plsc

NC, NS = 2, 16
NW = NC * NS
BATCH, SEQ, HID, VOCAB = 4, 8192, 2048, 256

N_SC_BATCH = 1
ROWS_SC = N_SC_BATCH * SEQ
RPW = ROWS_SC // NW
T = 8
NCHUNK = RPW // T

N_TC_BATCH = BATCH - N_SC_BATCH
ROWS_TC = N_TC_BATCH * SEQ
TB = 1024
LBLK = SEQ // TB


def _sc_embed(idx_hbm, emb_hbm, pos_hbm, out_hbm,
              idx_v, buf0, buf1, gbuf0, gbuf1,
              gsem0, gsem1, psem0, psem1, osem0, osem1):
    wid = lax.axis_index("s") * NC + lax.axis_index("c")
    base = pl.multiple_of(wid * RPW, RPW)
    pbase = lax.rem(base, SEQ)
    bufs = (buf0, buf1)
    gbufs = (gbuf0, gbuf1)
    gsems = (gsem0, gsem1)
    psems = (psem0, psem1)
    osems = (osem0, osem1)

    pltpu.sync_copy(idx_hbm.at[pl.ds(base, RPW)], idx_v)

    def launch(c, s):
        p0 = pl.multiple_of(pbase + c * T, T)

        @pl.when(c >= 2)
        def _():
            rp = pl.multiple_of(base + (c - 2) * T, T)
            pltpu.make_async_copy(bufs[s], out_hbm.at[pl.ds(rp, T)],
                                  osems[s]).wait()

        pltpu.async_copy(emb_hbm.at[idx_v.at[pl.ds(c * T, T)]],
                         gbufs[s], gsems[s])
        pltpu.async_copy(pos_hbm.at[pl.ds(p0, T)], bufs[s], psems[s])

    def finish(c, s):
        r0 = pl.multiple_of(base + c * T, T)
        p0 = pl.multiple_of(pbase + c * T, T)
        pltpu.make_async_copy(emb_hbm.at[idx_v.at[pl.ds(c * T, T)]],
                              gbufs[s], gsems[s]).wait()
        pltpu.make_async_copy(pos_hbm.at[pl.ds(p0, T)], bufs[s],
                              psems[s]).wait()

        @plsc.parallel_loop(0, HID, step=16, unroll=4)
        def add_body(j):
            for t in range(T):
                plsc.addupdate(bufs[s].at[t, pl.ds(j, 16)],
                               gbufs[s][t, pl.ds(j, 16)])

        pltpu.async_copy(bufs[s], out_hbm.at[pl.ds(r0, T)], osems[s])

    launch(0, 0)

    def body(i, carry):
        c = 2 * i
        launch(c + 1, 1)
        finish(c, 0)
        launch(c + 2, 0)
        finish(c + 1, 1)
        return carry

    lax.fori_loop(0, NCHUNK // 2 - 1, body, 0)

    c = NCHUNK - 2
    launch(c + 1, 1)
    finish(c, 0)
    finish(c + 1, 1)
    pltpu.make_async_copy(bufs[0], out_hbm.at[pl.ds(base + c * T, T)],
                          osems[0]).wait()
    pltpu.make_async_copy(bufs[1], out_hbm.at[pl.ds(base + (c + 1) * T, T)],
                          osems[1]).wait()


def _sc_call(idx_sc, emb_table, pos_table):
    mesh = plsc.VectorSubcoreMesh(
        core_axis_name="c", subcore_axis_name="s",
        num_cores=NC, num_subcores=NS)
    return pl.kernel(
        _sc_embed,
        out_type=jax.ShapeDtypeStruct((ROWS_SC, HID), jnp.float32),
        mesh=mesh,
        scratch_types=[
            pltpu.VMEM((RPW,), jnp.int32),
            pltpu.VMEM((T, HID), jnp.float32),
            pltpu.VMEM((T, HID), jnp.float32),
            pltpu.VMEM((T, HID), jnp.float32),
            pltpu.VMEM((T, HID), jnp.float32),
            pltpu.SemaphoreType.DMA,
            pltpu.SemaphoreType.DMA,
            pltpu.SemaphoreType.DMA,
            pltpu.SemaphoreType.DMA,
            pltpu.SemaphoreType.DMA,
            pltpu.SemaphoreType.DMA,
        ],
    )(idx_sc, emb_table, pos_table)


def _tc_embed(idx_ref, emb_ref, pos_ref, out_ref):
    ids = idx_ref[0, 0, :]
    onehot = (ids[:, None] ==
              lax.broadcasted_iota(jnp.int32, (1, VOCAB), 1)
              ).astype(jnp.bfloat16)
    tok = lax.dot_general(onehot, emb_ref[...],
                          (((1,), (0,)), ((), ())),
                          preferred_element_type=jnp.float32)
    out_ref[...] = tok + pos_ref[...]


def _tc_call(idx_tc, emb_bf16, pos_table):
    return pl.pallas_call(
        _tc_embed,
        grid=(LBLK, N_TC_BATCH),
        in_specs=[
            pl.BlockSpec((1, 1, TB), lambda lb, bb: (bb * LBLK + lb, 0, 0)),
            pl.BlockSpec((VOCAB, HID), lambda lb, bb: (0, 0)),
            pl.BlockSpec((TB, HID), lambda lb, bb: (lb, 0)),
        ],
        out_specs=pl.BlockSpec(
            (TB, HID),
            lambda lb, bb: ((N_SC_BATCH + bb) * LBLK + lb, 0)),
        out_shape=jax.ShapeDtypeStruct((BATCH * SEQ, HID), jnp.float32),
    )(idx_tc, emb_bf16, pos_table)


def kernel(input_bytes, emb_table, pos_table):
    idx = input_bytes.reshape(BATCH * SEQ).astype(jnp.int32)
    idx_sc = idx[:ROWS_SC]
    idx_tc = idx[ROWS_SC:].reshape(ROWS_TC // TB, 1, TB)
    emb_bf16 = emb_table.astype(jnp.bfloat16)
    sc_out = _sc_call(idx_sc, emb_table, pos_table)
    tc_full = _tc_call(idx_tc, emb_bf16, pos_table)
    out = lax.dynamic_update_slice(tc_full, sc_out, (0, 0))
    return out.reshape(BATCH, SEQ, HID)
